# baseline (speedup 1.0000x reference)
import functools

import jax
import jax.numpy as jnp
from jax import lax
from jax.experimental import pallas as pl
from jax.experimental.pallas import tpu as pltpu
from jax.experimental.pallas import tpu_sc as plsc

N = 10000
E = 320000
D = 128
H = 128
G = 64

NC = 2
NS = 16
NW = NC * NS

CHUNK = 128
TOTCH = 2560
EPAD = TOTCH * CHUNK
NROWS = 10240
ZSLAB = NROWS // NS
IB = 16
B0 = 10
B1 = 0


@functools.cache
def _make_sc_segment_sum(with_deg: bool):
  mesh = plsc.VectorSubcoreMesh(
      core_axis_name="c", subcore_axis_name="s", num_cores=NC,
      num_subcores=NS)

  out_type = [jax.ShapeDtypeStruct((NROWS, D), jnp.float32)]
  scratch = [
      pltpu.VMEM((IB, CHUNK), jnp.int32),
      pltpu.VMEM((IB, CHUNK), jnp.int32),
      pltpu.VMEM((2, CHUNK, D), jnp.float32),
      pltpu.VMEM_SHARED((NROWS, D), jnp.float32),
      pltpu.SemaphoreType.DMA,
      pltpu.SemaphoreType.DMA,
  ]
  if with_deg:
    out_type.append(jax.ShapeDtypeStruct((NROWS,), jnp.float32))
    scratch.append(pltpu.VMEM((CHUNK,), jnp.float32))
    scratch.append(pltpu.VMEM_SHARED((NROWS,), jnp.float32))

  def body(*refs):
    if with_deg:
      (table, src, dst, zslab, zdeg, ones_h, agg_out, deg_out,
       src_v, dst_v, rows_v, agg_sh, sem0, sem1, ones_v, deg_sh) = refs
    else:
      (table, src, dst, zslab, agg_out,
       src_v, dst_v, rows_v, agg_sh, sem0, sem1) = refs
    sems = (sem0, sem1)

    c = lax.axis_index("c")
    s = lax.axis_index("s")
    nblocks = jnp.where(c == 0, B0, B1)
    blockbase = s * nblocks + c * (NS * B0)

    @pl.when(c == 0)
    def _():
      pltpu.sync_copy(zslab, agg_sh.at[pl.ds(s * ZSLAB, ZSLAB)])
      if with_deg:
        pltpu.sync_copy(ones_h, ones_v)
        pltpu.sync_copy(zdeg, deg_sh.at[pl.ds(s * ZSLAB, ZSLAB)])
    plsc.subcore_barrier()

    def block(ib, carry):
      off = (blockbase + ib) * IB
      pltpu.sync_copy(src.at[pl.ds(off, IB)], src_v)
      pltpu.sync_copy(dst.at[pl.ds(off, IB)], dst_v)
      pend = pltpu.async_copy(table.at[src_v.at[0]], rows_v.at[0], sems[0])
      for j in range(IB):
        b = j % 2
        nxt = None
        if j + 1 < IB:
          nxt = pltpu.async_copy(table.at[src_v.at[j + 1]],
                                 rows_v.at[1 - b], sems[1 - b])
        pend.wait()
        pltpu.sync_copy(rows_v.at[b], agg_sh.at[dst_v.at[j]], add=True)
        if with_deg:
          pltpu.sync_copy(ones_v, deg_sh.at[dst_v.at[j]], add=True)
        pend = nxt
      return carry

    lax.fori_loop(0, nblocks, block, 0)
    plsc.subcore_barrier()

    @pl.when(c == 0)
    def _():
      pltpu.sync_copy(agg_sh.at[pl.ds(s * ZSLAB, ZSLAB)],
                      agg_out.at[pl.ds(s * ZSLAB, ZSLAB)])
      if with_deg:
        pltpu.sync_copy(deg_sh.at[pl.ds(s * ZSLAB, ZSLAB)],
                        deg_out.at[pl.ds(s * ZSLAB, ZSLAB)])

  return pl.kernel(body, out_type=out_type, mesh=mesh,
                   scratch_types=scratch)


ROWS = 1000
NBLK = N // ROWS


def _tc_layer_body(agg, deg, x, wl, wr, b, out):
  a = agg[...]
  d = deg[0, 0]
  inv = jnp.reshape(1.0 / jnp.maximum(d, 1.0), (ROWS, 1))
  h = jnp.dot(a * inv, wl[...], preferred_element_type=jnp.float32)
  h = h + jnp.dot(x[...], wr[...], preferred_element_type=jnp.float32)
  out[...] = jnp.maximum(h + b[...], 0.0)


_tc_layer = pl.pallas_call(
    _tc_layer_body,
    grid=(NBLK,),
    in_specs=[
        pl.BlockSpec((ROWS, D), lambda i: (i, 0)),
        pl.BlockSpec((1, 1, ROWS), lambda i: (i, 0, 0)),
        pl.BlockSpec((ROWS, D), lambda i: (i, 0)),
        pl.BlockSpec((D, H), lambda i: (0, 0)),
        pl.BlockSpec((D, H), lambda i: (0, 0)),
        pl.BlockSpec((1, H), lambda i: (0, 0)),
    ],
    out_specs=pl.BlockSpec((ROWS, H), lambda i: (i, 0)),
    out_shape=jax.ShapeDtypeStruct((N, H), jnp.float32),
)


def _tc_layer2_body(agg, deg, x, batch, oh, wl, wr, b, w1, b1, w2, b2,
                    out, pooled):
  i = pl.program_id(0)
  a = agg[...]
  d = deg[0, 0]
  inv = jnp.reshape(1.0 / jnp.maximum(d, 1.0), (ROWS, 1))
  h = jnp.dot(a * inv, wl[...], preferred_element_type=jnp.float32)
  h = h + jnp.dot(x[...], wr[...], preferred_element_type=jnp.float32)
  h = jnp.maximum(h + b[...], 0.0)

  @pl.when(i == 0)
  def _():
    pooled[...] = jnp.zeros_like(pooled)

  bb = batch[0, 0]
  glo = jnp.min(bb)
  ghi = jnp.max(bb)

  def seg_step(g):
    def _do():
      col = oh[:, g:g + 1]
      v = jnp.max(h * col, axis=0, keepdims=True)
      pooled[g:g + 1, :] = jnp.maximum(pooled[g:g + 1, :], v)
    pl.when(jnp.logical_and(g >= glo, g <= ghi))(_do)

  for g in range(G):
    seg_step(g)

  @pl.when(i == NBLK - 1)
  def _():
    p = pooled[...]
    g1 = jnp.dot(p, w1[...], preferred_element_type=jnp.float32)
    g1 = jnp.maximum(g1 + b1[...], 0.0)
    g2 = jnp.dot(g1, w2[...], preferred_element_type=jnp.float32)
    out[...] = g2 + b2[...]


_tc_layer2 = pl.pallas_call(
    _tc_layer2_body,
    grid=(NBLK,),
    in_specs=[
        pl.BlockSpec((ROWS, D), lambda i: (i, 0)),
        pl.BlockSpec((1, 1, ROWS), lambda i: (i, 0, 0)),
        pl.BlockSpec((ROWS, D), lambda i: (i, 0)),
        pl.BlockSpec((1, 1, ROWS), lambda i: (i, 0, 0)),
        pl.BlockSpec((ROWS, G), lambda i: (i, 0)),
        pl.BlockSpec((D, H), lambda i: (0, 0)),
        pl.BlockSpec((D, H), lambda i: (0, 0)),
        pl.BlockSpec((1, H), lambda i: (0, 0)),
        pl.BlockSpec((H, H), lambda i: (0, 0)),
        pl.BlockSpec((1, H), lambda i: (0, 0)),
        pl.BlockSpec((H, H), lambda i: (0, 0)),
        pl.BlockSpec((1, H), lambda i: (0, 0)),
    ],
    out_specs=pl.BlockSpec((G, H), lambda i: (0, 0)),
    out_shape=jax.ShapeDtypeStruct((G, H), jnp.float32),
    scratch_shapes=[pltpu.VMEM((G, H), jnp.float32)],
)


def _fold_bn(wl, wr, b, gamma, beta, rm, rv):
  s = gamma / jnp.sqrt(rv + 1e-5)
  return wl * s[None, :], wr * s[None, :], ((b - rm) * s + beta)[None, :]


def kernel(x, edge_index, batch, Wl1, Wr1, b1, gamma1, beta1, rm1, rv1,
           Wl2, Wr2, b2, gamma2, beta2, rm2, rv2, Wlin1, blin1, Wlin2,
           blin2):
  pad = EPAD - E
  src = jnp.concatenate([edge_index[0], jnp.zeros((pad,), jnp.int32)])
  dpad = N + (jnp.arange(pad, dtype=jnp.int32) % (NROWS - N))
  dst = jnp.concatenate([edge_index[1], dpad])
  src3 = src.reshape(TOTCH, CHUNK)
  dst3 = dst.reshape(TOTCH, CHUNK)

  zslab = jnp.zeros((ZSLAB, D), jnp.float32)
  zdeg = jnp.zeros((ZSLAB,), jnp.float32)
  ones = jnp.ones((CHUNK,), jnp.float32)

  wl1, wr1, bb1 = _fold_bn(Wl1, Wr1, b1, gamma1, beta1, rm1, rv1)
  wl2, wr2, bb2 = _fold_bn(Wl2, Wr2, b2, gamma2, beta2, rm2, rv2)
  w2p = jnp.zeros((H, H), jnp.float32).at[:, :2].set(Wlin2)
  b2p = jnp.zeros((1, H), jnp.float32).at[0, :2].set(blin2)

  agg1, deg = _make_sc_segment_sum(True)(x, src3, dst3, zslab, zdeg, ones)
  degb = deg[:N].reshape(NBLK, 1, ROWS)
  h1 = _tc_layer(agg1, degb, x, wl1, wr1, bb1)
  (agg2,) = _make_sc_segment_sum(False)(h1, src3, dst3, zslab)
  onehot = (batch[:, None] == jnp.arange(G)[None, :]).astype(jnp.float32)
  out = _tc_layer2(agg2, degb, h1, batch.reshape(NBLK, 1, ROWS), onehot,
                   wl2, wr2, bb2, Wlin1, blin1[None, :], w2p, b2p)
  return out[:, :2]


# speedup vs baseline: 1.3048x; 1.3048x over previous
import functools

import jax
import jax.numpy as jnp
from jax import lax
from jax.experimental import pallas as pl
from jax.experimental.pallas import tpu as pltpu
from jax.experimental.pallas import tpu_sc as plsc

N = 10000
E = 320000
D = 128
H = 128
G = 64

NC = 2
NS = 16

CHUNK = 128
TOTCH = 2560
EPAD = TOTCH * CHUNK
NROWS = 10240
ZSLAB = NROWS // NS
IB = 16
B0 = 8
B1 = 2


@functools.cache
def _make_sc_segment_sum(with_deg: bool):
  mesh = plsc.VectorSubcoreMesh(
      core_axis_name="c", subcore_axis_name="s", num_cores=NC,
      num_subcores=NS)

  out_type = [jax.ShapeDtypeStruct((NC, NROWS, D), jnp.float32)]
  scratch = [
      pltpu.VMEM((IB, CHUNK), jnp.int32),
      pltpu.VMEM((IB, CHUNK), jnp.int32),
      pltpu.VMEM((2, CHUNK, D), jnp.float32),
      pltpu.VMEM_SHARED((NROWS, D), jnp.float32),
      pltpu.SemaphoreType.DMA,
      pltpu.SemaphoreType.DMA,
  ]
  if with_deg:
    out_type.append(jax.ShapeDtypeStruct((NC * NROWS,), jnp.float32))
    scratch.append(pltpu.VMEM((CHUNK,), jnp.float32))
    scratch.append(pltpu.VMEM_SHARED((NROWS,), jnp.float32))

  def body(*refs):
    if with_deg:
      (table, src, dst, agg_out, deg_out,
       src_v, dst_v, rows_v, agg_sh, sem0, sem1, ones_v, deg_sh) = refs
    else:
      (table, src, dst, agg_out,
       src_v, dst_v, rows_v, agg_sh, sem0, sem1) = refs
    sems = (sem0, sem1)

    c = lax.axis_index("c")
    s = lax.axis_index("s")
    nblocks = jnp.where(c == 0, B0, B1)
    blockbase = s * nblocks + c * (NS * B0)

    zv = jnp.zeros((16,), jnp.float32)

    def zrow(r, carry):
      for k in range(D // 16):
        rows_v[0, r, pl.ds(16 * k, 16)] = zv
      return carry

    lax.fori_loop(0, CHUNK, zrow, 0)
    for t in range(ZSLAB // CHUNK):
      pltpu.sync_copy(rows_v.at[0],
                      agg_sh.at[pl.ds(s * ZSLAB + t * CHUNK, CHUNK)])
    if with_deg:
      def zdrow(r, carry):
        ones_v[pl.ds(16 * r, 16)] = zv
        return carry
      lax.fori_loop(0, CHUNK // 16, zdrow, 0)
      for t in range(ZSLAB // CHUNK):
        pltpu.sync_copy(ones_v,
                        deg_sh.at[pl.ds(s * ZSLAB + t * CHUNK, CHUNK)])
      ov = jnp.ones((16,), jnp.float32)

      def orow(r, carry):
        ones_v[pl.ds(16 * r, 16)] = ov
        return carry
      lax.fori_loop(0, CHUNK // 16, orow, 0)
    plsc.subcore_barrier()

    def block(ib, carry):
      off = (blockbase + ib) * IB
      pltpu.sync_copy(src.at[pl.ds(off, IB)], src_v)
      pltpu.sync_copy(dst.at[pl.ds(off, IB)], dst_v)
      pend = pltpu.async_copy(table.at[src_v.at[0]], rows_v.at[0], sems[0])
      for j in range(IB):
        b = j % 2
        nxt = None
        if j + 1 < IB:
          nxt = pltpu.async_copy(table.at[src_v.at[j + 1]],
                                 rows_v.at[1 - b], sems[1 - b])
        pend.wait()
        pltpu.sync_copy(rows_v.at[b], agg_sh.at[dst_v.at[j]], add=True)
        if with_deg:
          pltpu.sync_copy(ones_v, deg_sh.at[dst_v.at[j]], add=True)
        pend = nxt
      return carry

    lax.fori_loop(0, nblocks, block, 0)
    plsc.subcore_barrier()

    pltpu.sync_copy(agg_sh.at[pl.ds(s * ZSLAB, ZSLAB)],
                    agg_out.at[c, pl.ds(s * ZSLAB, ZSLAB)])
    if with_deg:
      pltpu.sync_copy(deg_sh.at[pl.ds(s * ZSLAB, ZSLAB)],
                      deg_out.at[pl.ds(c * NROWS + s * ZSLAB, ZSLAB)])

  return pl.kernel(body, out_type=out_type, mesh=mesh,
                   scratch_types=scratch)


ROWS = 1000
NBLK = N // ROWS


def _tc_layer_body(agg, deg, x, wl, wr, b, out):
  a = agg[0] + agg[1]
  d = deg[0, 0, 0] + deg[0, 1, 0]
  inv = jnp.reshape(1.0 / jnp.maximum(d, 1.0), (ROWS, 1))
  h = jnp.dot(a * inv, wl[...], preferred_element_type=jnp.float32)
  h = h + jnp.dot(x[...], wr[...], preferred_element_type=jnp.float32)
  out[...] = jnp.maximum(h + b[...], 0.0)


_tc_layer = pl.pallas_call(
    _tc_layer_body,
    grid=(NBLK,),
    in_specs=[
        pl.BlockSpec((NC, ROWS, D), lambda i: (0, i, 0)),
        pl.BlockSpec((1, NC, 1, ROWS), lambda i: (i, 0, 0, 0)),
        pl.BlockSpec((ROWS, D), lambda i: (i, 0)),
        pl.BlockSpec((D, H), lambda i: (0, 0)),
        pl.BlockSpec((D, H), lambda i: (0, 0)),
        pl.BlockSpec((1, H), lambda i: (0, 0)),
    ],
    out_specs=pl.BlockSpec((ROWS, H), lambda i: (i, 0)),
    out_shape=jax.ShapeDtypeStruct((N, H), jnp.float32),
)


def _tc_layer2_body(agg, deg, x, batch, oh, wl, wr, b, w1, b1, w2, b2,
                    out, pooled):
  i = pl.program_id(0)
  a = agg[0] + agg[1]
  d = deg[0, 0, 0] + deg[0, 1, 0]
  inv = jnp.reshape(1.0 / jnp.maximum(d, 1.0), (ROWS, 1))
  h = jnp.dot(a * inv, wl[...], preferred_element_type=jnp.float32)
  h = h + jnp.dot(x[...], wr[...], preferred_element_type=jnp.float32)
  h = jnp.maximum(h + b[...], 0.0)

  @pl.when(i == 0)
  def _():
    pooled[...] = jnp.zeros_like(pooled)

  bb = batch[0, 0]
  glo = jnp.min(bb)
  ghi = jnp.max(bb)

  def seg_step(g):
    def _do():
      col = oh[:, g:g + 1]
      v = jnp.max(h * col, axis=0, keepdims=True)
      pooled[g:g + 1, :] = jnp.maximum(pooled[g:g + 1, :], v)
    pl.when(jnp.logical_and(g >= glo, g <= ghi))(_do)

  for g in range(G):
    seg_step(g)

  @pl.when(i == NBLK - 1)
  def _():
    p = pooled[...]
    g1 = jnp.dot(p, w1[...], preferred_element_type=jnp.float32)
    g1 = jnp.maximum(g1 + b1[...], 0.0)
    g2 = jnp.dot(g1, w2[...], preferred_element_type=jnp.float32)
    out[...] = g2 + b2[...]


_tc_layer2 = pl.pallas_call(
    _tc_layer2_body,
    grid=(NBLK,),
    in_specs=[
        pl.BlockSpec((NC, ROWS, D), lambda i: (0, i, 0)),
        pl.BlockSpec((1, NC, 1, ROWS), lambda i: (i, 0, 0, 0)),
        pl.BlockSpec((ROWS, D), lambda i: (i, 0)),
        pl.BlockSpec((1, 1, ROWS), lambda i: (i, 0, 0)),
        pl.BlockSpec((ROWS, G), lambda i: (i, 0)),
        pl.BlockSpec((D, H), lambda i: (0, 0)),
        pl.BlockSpec((D, H), lambda i: (0, 0)),
        pl.BlockSpec((1, H), lambda i: (0, 0)),
        pl.BlockSpec((H, H), lambda i: (0, 0)),
        pl.BlockSpec((1, H), lambda i: (0, 0)),
        pl.BlockSpec((H, H), lambda i: (0, 0)),
        pl.BlockSpec((1, H), lambda i: (0, 0)),
    ],
    out_specs=pl.BlockSpec((G, H), lambda i: (0, 0)),
    out_shape=jax.ShapeDtypeStruct((G, H), jnp.float32),
    scratch_shapes=[pltpu.VMEM((G, H), jnp.float32)],
)


def _fold_bn(wl, wr, b, gamma, beta, rm, rv):
  s = gamma / jnp.sqrt(rv + 1e-5)
  return wl * s[None, :], wr * s[None, :], ((b - rm) * s + beta)[None, :]


def kernel(x, edge_index, batch, Wl1, Wr1, b1, gamma1, beta1, rm1, rv1,
           Wl2, Wr2, b2, gamma2, beta2, rm2, rv2, Wlin1, blin1, Wlin2,
           blin2):
  pad = EPAD - E
  src = jnp.concatenate([edge_index[0], jnp.zeros((pad,), jnp.int32)])
  dpad = N + (jnp.arange(pad, dtype=jnp.int32) % (NROWS - N))
  dst = jnp.concatenate([edge_index[1], dpad])
  src3 = src.reshape(TOTCH, CHUNK)
  dst3 = dst.reshape(TOTCH, CHUNK)

  wl1, wr1, bb1 = _fold_bn(Wl1, Wr1, b1, gamma1, beta1, rm1, rv1)
  wl2, wr2, bb2 = _fold_bn(Wl2, Wr2, b2, gamma2, beta2, rm2, rv2)
  w2p = jnp.zeros((H, H), jnp.float32).at[:, :2].set(Wlin2)
  b2p = jnp.zeros((1, H), jnp.float32).at[0, :2].set(blin2)

  agg1, deg = _make_sc_segment_sum(True)(x, src3, dst3)
  degb = deg.reshape(NC, NROWS)[:, :N].reshape(NC, NBLK, ROWS)
  degb = degb.transpose(1, 0, 2).reshape(NBLK, NC, 1, ROWS)
  h1 = _tc_layer(agg1, degb, x, wl1, wr1, bb1)
  (agg2,) = _make_sc_segment_sum(False)(h1, src3, dst3)
  onehot = (batch[:, None] == jnp.arange(G)[None, :]).astype(jnp.float32)
  out = _tc_layer2(agg2, degb, h1, batch.reshape(NBLK, 1, ROWS), onehot,
                   wl2, wr2, bb2, Wlin1, blin1[None, :], w2p, b2p)
  return out[:, :2]


# speedup vs baseline: 1.3133x; 1.0065x over previous
import functools

import jax
import jax.numpy as jnp
from jax import lax
from jax.experimental import pallas as pl
from jax.experimental.pallas import tpu as pltpu
from jax.experimental.pallas import tpu_sc as plsc

N = 10000
E = 320000
D = 128
H = 128
G = 64

NC = 2
NS = 16

CHUNK = 128
TOTCH = 2560
EPAD = TOTCH * CHUNK
NROWS = 10240
ZSLAB = NROWS // NS
IB = 8
B0 = 16
B1 = 4


@functools.cache
def _make_sc_segment_sum(with_deg: bool):
  mesh = plsc.VectorSubcoreMesh(
      core_axis_name="c", subcore_axis_name="s", num_cores=NC,
      num_subcores=NS)

  out_type = [jax.ShapeDtypeStruct((NC, NROWS, D), jnp.float32)]
  scratch = [
      pltpu.VMEM((IB, CHUNK), jnp.int32),
      pltpu.VMEM((IB, CHUNK), jnp.int32),
      pltpu.VMEM((2, CHUNK, D), jnp.float32),
      pltpu.VMEM_SHARED((NROWS, D), jnp.float32),
      pltpu.SemaphoreType.DMA,
      pltpu.SemaphoreType.DMA,
  ]
  if with_deg:
    out_type.append(jax.ShapeDtypeStruct((NC * NROWS,), jnp.float32))
    scratch.append(pltpu.VMEM((CHUNK,), jnp.float32))
    scratch.append(pltpu.VMEM_SHARED((NROWS,), jnp.float32))

  def body(*refs):
    if with_deg:
      (table, src, dst, agg_out, deg_out,
       src_v, dst_v, rows_v, agg_sh, sem0, sem1, ones_v, deg_sh) = refs
    else:
      (table, src, dst, agg_out,
       src_v, dst_v, rows_v, agg_sh, sem0, sem1) = refs
    sems = (sem0, sem1)

    c = lax.axis_index("c")
    s = lax.axis_index("s")
    nblocks = jnp.where(c == 0, B0, B1)
    blockbase = s * nblocks + c * (NS * B0)

    zv = jnp.zeros((16,), jnp.float32)

    def zrow(r, carry):
      for k in range(D // 16):
        rows_v[0, r, pl.ds(16 * k, 16)] = zv
      return carry

    lax.fori_loop(0, CHUNK, zrow, 0)
    for t in range(ZSLAB // CHUNK):
      pltpu.sync_copy(rows_v.at[0],
                      agg_sh.at[pl.ds(s * ZSLAB + t * CHUNK, CHUNK)])
    if with_deg:
      def zdrow(r, carry):
        ones_v[pl.ds(16 * r, 16)] = zv
        return carry
      lax.fori_loop(0, CHUNK // 16, zdrow, 0)
      for t in range(ZSLAB // CHUNK):
        pltpu.sync_copy(ones_v,
                        deg_sh.at[pl.ds(s * ZSLAB + t * CHUNK, CHUNK)])
      ov = jnp.ones((16,), jnp.float32)

      def orow(r, carry):
        ones_v[pl.ds(16 * r, 16)] = ov
        return carry
      lax.fori_loop(0, CHUNK // 16, orow, 0)
    plsc.subcore_barrier()

    def block(ib, carry):
      off = (blockbase + ib) * IB
      pltpu.sync_copy(src.at[pl.ds(off, IB)], src_v)
      pltpu.sync_copy(dst.at[pl.ds(off, IB)], dst_v)
      pend = pltpu.async_copy(table.at[src_v.at[0]], rows_v.at[0], sems[0])
      for j in range(IB):
        b = j % 2
        nxt = None
        if j + 1 < IB:
          nxt = pltpu.async_copy(table.at[src_v.at[j + 1]],
                                 rows_v.at[1 - b], sems[1 - b])
        pend.wait()
        pltpu.sync_copy(rows_v.at[b], agg_sh.at[dst_v.at[j]], add=True)
        if with_deg:
          pltpu.sync_copy(ones_v, deg_sh.at[dst_v.at[j]], add=True)
        pend = nxt
      return carry

    lax.fori_loop(0, nblocks, block, 0)
    plsc.subcore_barrier()

    pltpu.sync_copy(agg_sh.at[pl.ds(s * ZSLAB, ZSLAB)],
                    agg_out.at[c, pl.ds(s * ZSLAB, ZSLAB)])
    if with_deg:
      pltpu.sync_copy(deg_sh.at[pl.ds(s * ZSLAB, ZSLAB)],
                      deg_out.at[pl.ds(c * NROWS + s * ZSLAB, ZSLAB)])

  return pl.kernel(body, out_type=out_type, mesh=mesh,
                   scratch_types=scratch)


ROWS = 1000
NBLK = N // ROWS


def _tc_layer_body(agg, deg, x, wl, wr, b, out):
  a = agg[0] + agg[1]
  d = deg[0, 0, 0] + deg[0, 1, 0]
  inv = jnp.reshape(1.0 / jnp.maximum(d, 1.0), (ROWS, 1))
  h = jnp.dot(a * inv, wl[...], preferred_element_type=jnp.float32)
  h = h + jnp.dot(x[...], wr[...], preferred_element_type=jnp.float32)
  out[...] = jnp.maximum(h + b[...], 0.0)


_tc_layer = pl.pallas_call(
    _tc_layer_body,
    grid=(NBLK,),
    in_specs=[
        pl.BlockSpec((NC, ROWS, D), lambda i: (0, i, 0)),
        pl.BlockSpec((1, NC, 1, ROWS), lambda i: (i, 0, 0, 0)),
        pl.BlockSpec((ROWS, D), lambda i: (i, 0)),
        pl.BlockSpec((D, H), lambda i: (0, 0)),
        pl.BlockSpec((D, H), lambda i: (0, 0)),
        pl.BlockSpec((1, H), lambda i: (0, 0)),
    ],
    out_specs=pl.BlockSpec((ROWS, H), lambda i: (i, 0)),
    out_shape=jax.ShapeDtypeStruct((N, H), jnp.float32),
)


def _tc_layer2_body(agg, deg, x, batch, oh, wl, wr, b, w1, b1, w2, b2,
                    out, pooled):
  i = pl.program_id(0)
  a = agg[0] + agg[1]
  d = deg[0, 0, 0] + deg[0, 1, 0]
  inv = jnp.reshape(1.0 / jnp.maximum(d, 1.0), (ROWS, 1))
  h = jnp.dot(a * inv, wl[...], preferred_element_type=jnp.float32)
  h = h + jnp.dot(x[...], wr[...], preferred_element_type=jnp.float32)
  h = jnp.maximum(h + b[...], 0.0)

  @pl.when(i == 0)
  def _():
    pooled[...] = jnp.zeros_like(pooled)

  bb = batch[0, 0]
  glo = jnp.min(bb)
  ghi = jnp.max(bb)

  def seg_step(g):
    def _do():
      col = oh[:, g:g + 1]
      v = jnp.max(h * col, axis=0, keepdims=True)
      pooled[g:g + 1, :] = jnp.maximum(pooled[g:g + 1, :], v)
    pl.when(jnp.logical_and(g >= glo, g <= ghi))(_do)

  for g in range(G):
    seg_step(g)

  @pl.when(i == NBLK - 1)
  def _():
    p = pooled[...]
    g1 = jnp.dot(p, w1[...], preferred_element_type=jnp.float32)
    g1 = jnp.maximum(g1 + b1[...], 0.0)
    g2 = jnp.dot(g1, w2[...], preferred_element_type=jnp.float32)
    out[...] = g2 + b2[...]


_tc_layer2 = pl.pallas_call(
    _tc_layer2_body,
    grid=(NBLK,),
    in_specs=[
        pl.BlockSpec((NC, ROWS, D), lambda i: (0, i, 0)),
        pl.BlockSpec((1, NC, 1, ROWS), lambda i: (i, 0, 0, 0)),
        pl.BlockSpec((ROWS, D), lambda i: (i, 0)),
        pl.BlockSpec((1, 1, ROWS), lambda i: (i, 0, 0)),
        pl.BlockSpec((ROWS, G), lambda i: (i, 0)),
        pl.BlockSpec((D, H), lambda i: (0, 0)),
        pl.BlockSpec((D, H), lambda i: (0, 0)),
        pl.BlockSpec((1, H), lambda i: (0, 0)),
        pl.BlockSpec((H, H), lambda i: (0, 0)),
        pl.BlockSpec((1, H), lambda i: (0, 0)),
        pl.BlockSpec((H, H), lambda i: (0, 0)),
        pl.BlockSpec((1, H), lambda i: (0, 0)),
    ],
    out_specs=pl.BlockSpec((G, H), lambda i: (0, 0)),
    out_shape=jax.ShapeDtypeStruct((G, H), jnp.float32),
    scratch_shapes=[pltpu.VMEM((G, H), jnp.float32)],
)


def _fold_bn(wl, wr, b, gamma, beta, rm, rv):
  s = gamma / jnp.sqrt(rv + 1e-5)
  return wl * s[None, :], wr * s[None, :], ((b - rm) * s + beta)[None, :]


def kernel(x, edge_index, batch, Wl1, Wr1, b1, gamma1, beta1, rm1, rv1,
           Wl2, Wr2, b2, gamma2, beta2, rm2, rv2, Wlin1, blin1, Wlin2,
           blin2):
  pad = EPAD - E
  src = jnp.concatenate([edge_index[0], jnp.zeros((pad,), jnp.int32)])
  dpad = N + (jnp.arange(pad, dtype=jnp.int32) % (NROWS - N))
  dst = jnp.concatenate([edge_index[1], dpad])
  src3 = src.reshape(TOTCH, CHUNK)
  dst3 = dst.reshape(TOTCH, CHUNK)

  wl1, wr1, bb1 = _fold_bn(Wl1, Wr1, b1, gamma1, beta1, rm1, rv1)
  wl2, wr2, bb2 = _fold_bn(Wl2, Wr2, b2, gamma2, beta2, rm2, rv2)
  w2p = jnp.zeros((H, H), jnp.float32).at[:, :2].set(Wlin2)
  b2p = jnp.zeros((1, H), jnp.float32).at[0, :2].set(blin2)

  agg1, deg = _make_sc_segment_sum(True)(x, src3, dst3)
  degb = deg.reshape(NC, NROWS)[:, :N].reshape(NC, NBLK, ROWS)
  degb = degb.transpose(1, 0, 2).reshape(NBLK, NC, 1, ROWS)
  h1 = _tc_layer(agg1, degb, x, wl1, wr1, bb1)
  (agg2,) = _make_sc_segment_sum(False)(h1, src3, dst3)
  onehot = (batch[:, None] == jnp.arange(G)[None, :]).astype(jnp.float32)
  out = _tc_layer2(agg2, degb, h1, batch.reshape(NBLK, 1, ROWS), onehot,
                   wl2, wr2, bb2, Wlin1, blin1[None, :], w2p, b2p)
  return out[:, :2]
